# baseline (speedup 1.0000x reference)
import functools

import jax
import jax.numpy as jnp
import numpy as np
from jax import lax
from jax.experimental import pallas as pl
from jax.experimental.pallas import tpu as pltpu
from jax.experimental.pallas import tpu_sc as plsc

F = 26
D = 16
B = 16384
EO = F * D
FIELD_DIM = 100000
BN_SCALE = 1.0 / (1.0 + 1e-5) ** 0.5

NUM_TILES = 32
SAMP_PER_TILE = B // NUM_TILES
CS = 128
NSUB = SAMP_PER_TILE // CS


def _sc_gather(x_t, emb_table, lin_flat):
    mesh = plsc.VectorSubcoreMesh(core_axis_name="c", subcore_axis_name="s")

    @functools.partial(
        pl.kernel,
        mesh=mesh,
        compiler_params=pltpu.CompilerParams(use_tc_tiling_on_sc=False),
        out_type=(
            jax.ShapeDtypeStruct((B, EO), jnp.float32),
            jax.ShapeDtypeStruct((B,), jnp.float32),
        ),
        scratch_types=[
            pltpu.VMEM((F, CS), jnp.int32),
            pltpu.VMEM((F, CS), jnp.int32),
            pltpu.VMEM((F, CS, D), jnp.float32),
            pltpu.VMEM((F, CS), jnp.float32),
            pltpu.VMEM((CS,), jnp.float32),
            pltpu.SemaphoreType.DMA,
            pltpu.SemaphoreType.DMA,
            pltpu.SemaphoreType.DMA,
        ],
    )
    def body(x_hbm, emb_hbm, lin_hbm, emb_out, lin_out,
             xbuf, idxbuf, embbuf, linbuf, accbuf, sem_e, sem_l, sem_o):
        wid = lax.axis_index("s") * 2 + lax.axis_index("c")

        def sub_body(sub, carry):
            base = wid * SAMP_PER_TILE + sub * CS
            pltpu.sync_copy(x_hbm.at[:, pl.ds(base, CS)], xbuf)
            for f in range(F):
                off = jnp.full((16,), f * FIELD_DIM, jnp.int32)
                for j in range(CS // 16):
                    idxbuf[f, pl.ds(16 * j, 16)] = (
                        xbuf[f, pl.ds(16 * j, 16)] + off)
            ces = []
            cls = []
            for f in range(F):
                ces.append(pltpu.async_copy(
                    emb_hbm.at[idxbuf.at[f]], embbuf.at[f], sem_e))
                cls.append(pltpu.async_copy(
                    lin_hbm.at[idxbuf.at[f]], linbuf.at[f], sem_l))
            for c in cls:
                c.wait()
            for j in range(CS // 16):
                acc = linbuf[0, pl.ds(16 * j, 16)]
                for f in range(1, F):
                    acc = acc + linbuf[f, pl.ds(16 * j, 16)]
                accbuf[pl.ds(16 * j, 16)] = acc
            pltpu.sync_copy(accbuf, lin_out.at[pl.ds(base, CS)])
            for c in ces:
                c.wait()
            cos = []
            for f in range(F):
                cos.append(pltpu.async_copy(
                    embbuf.at[f],
                    emb_out.at[pl.ds(base, CS), pl.ds(D * f, D)], sem_o))
            for c in cos:
                c.wait()
            return carry

        lax.fori_loop(0, NSUB, sub_body, 0)

    return body(x_t, emb_table, lin_flat)


def _tc_body(emb_ref, lin_ref, w1_ref, b1_ref, w2_ref, b2_ref, w3_ref,
             ssum_ref, b3_ref, out_ref):
    e = emb_ref[...]
    h1 = jnp.dot(e, w1_ref[...], preferred_element_type=jnp.float32)
    s = jnp.dot(e, ssum_ref[...], preferred_element_type=jnp.float32)
    q = jnp.sum(e * e, axis=1)
    fm = 0.5 * (jnp.sum(s * s, axis=1) - q)
    h1 = jnp.maximum(h1 + b1_ref[...], 0.0)
    h2 = jnp.dot(h1, w2_ref[...], preferred_element_type=jnp.float32)
    h2 = jnp.maximum(h2 + b2_ref[...], 0.0)
    oc = jnp.sum(h2 * w3_ref[...], axis=1)
    out_ref[...] = lin_ref[...] + fm + oc + b3_ref[0]


def _tc_head(emb, lin, w1f, b1f, w2f, b2f, w3r, ssum, b3s, bm):
    grid = (B // bm,)
    return pl.pallas_call(
        _tc_body,
        grid=grid,
        in_specs=[
            pl.BlockSpec((bm, EO), lambda i: (i, 0)),
            pl.BlockSpec((bm,), lambda i: (i,)),
            pl.BlockSpec((EO, 128), lambda i: (0, 0)),
            pl.BlockSpec((1, 128), lambda i: (0, 0)),
            pl.BlockSpec((128, 64), lambda i: (0, 0)),
            pl.BlockSpec((1, 64), lambda i: (0, 0)),
            pl.BlockSpec((1, 64), lambda i: (0, 0)),
            pl.BlockSpec((EO, D), lambda i: (0, 0)),
            pl.BlockSpec(memory_space=pltpu.SMEM),
        ],
        out_specs=pl.BlockSpec((bm,), lambda i: (i,)),
        out_shape=jax.ShapeDtypeStruct((B,), jnp.float32),
    )(emb, lin, w1f, b1f, w2f, b2f, w3r, ssum, b3s)


def kernel(x, emb_table, lin_w, lin_b, W1, b1, g1, be1, W2, b2, g2, be2,
           W3, b3):
    x_t = jnp.asarray(x.T, jnp.int32)
    lin_flat = lin_w[:, 0]

    scale1 = BN_SCALE * g1
    w1f = W1 * scale1[None, :]
    b1f = (b1 * scale1 + be1)[None, :]
    scale2 = BN_SCALE * g2
    w2f = W2 * scale2[None, :]
    b2f = (b2 * scale2 + be2)[None, :]
    w3r = W3.reshape(1, 64)
    ssum = jnp.asarray(np.tile(np.eye(D, dtype=np.float32), (F, 1)))
    b3s = (b3 + lin_b)

    emb, linear = _sc_gather(x_t, emb_table, lin_flat)

    bm = 2048
    return _tc_head(emb, linear, w1f, b1f, w2f, b2f, w3r, ssum, b3s, bm)
